# Initial kernel scaffold; baseline (speedup 1.0000x reference)
import jax
import jax.numpy as jnp
from jax.experimental import pallas as pl


def kernel(input_ids, word_emb, pos_emb, type_emb, gamma, beta):
    raise NotImplementedError("write your pallas kernel here")


# speedup vs baseline: 3.6228x; 3.6228x over previous
import functools

import jax
import jax.numpy as jnp
from jax import lax
from jax.experimental import pallas as pl
from jax.experimental.pallas import tpu as pltpu
from jax.experimental.pallas import tpu_sc as plsc

VOCAB = 100000
EMBED = 128
S = 200
B = 1024
EPS = 1e-5

NC, NS, L = 2, 16, 16
NW = NC * NS
ROWS_PER_W = B // NW
NJ = EMBED // L

_GATHER_DNUMS = lax.GatherDimensionNumbers(
    offset_dims=(), collapsed_slice_dims=(0,), start_index_map=(0,))


def _allsum(v):
    lanes = lax.iota(jnp.int32, L)
    for k in (1, 2, 4, 8):
        idx = (lanes ^ k)[:, None]
        v = v + lax.gather(v, idx, dimension_numbers=_GATHER_DNUMS,
                           slice_sizes=(1,),
                           mode=lax.GatherScatterMode.PROMISE_IN_BOUNDS)
    return v


def _sc_kernel(ids_hbm, table_hbm, pos_hbm, type_hbm, gamma_hbm, beta_hbm,
               out_hbm, ids_v, rows_v, bias_v, t_v, g_v, be_v, sem):
    wid = lax.axis_index("s") * NC + lax.axis_index("c")

    pltpu.sync_copy(pos_hbm.at[pl.ds(0, S)], bias_v)
    pltpu.sync_copy(type_hbm.at[pl.ds(0, 1)], t_v)
    pltpu.sync_copy(gamma_hbm, g_v)
    pltpu.sync_copy(beta_hbm, be_v)

    def add_type(r, carry):
        for j in range(NJ):
            sl = pl.ds(j * L, L)
            bias_v[r, sl] = bias_v[r, sl] + t_v[0, sl]
        return carry

    lax.fori_loop(0, S, add_type, 0)

    def chunk(c, carry):
        b = wid * ROWS_PER_W + c
        pltpu.sync_copy(ids_hbm.at[pl.ds(2 * b, 2)], ids_v)
        cp0 = pltpu.async_copy(table_hbm.at[ids_v.at[0]],
                               rows_v.at[pl.ds(0, 100)], sem)
        cp1 = pltpu.async_copy(table_hbm.at[ids_v.at[1]],
                               rows_v.at[pl.ds(100, 100)], sem)
        cp0.wait()
        cp1.wait()

        def token(i, icarry):
            xb = []
            for j in range(NJ):
                sl = pl.ds(j * L, L)
                xb.append(rows_v[i, sl] + bias_v[i, sl])
            ssum = xb[0]
            for j in range(1, NJ):
                ssum = ssum + xb[j]
            mean = _allsum(ssum) * (1.0 / EMBED)
            ssq = xb[0] * xb[0]
            for j in range(1, NJ):
                ssq = ssq + xb[j] * xb[j]
            var = _allsum(ssq) * (1.0 / EMBED) - mean * mean
            vv = var + EPS
            yi = jnp.int32(0x5F3759DF) - (lax.bitcast_convert_type(vv, jnp.int32) >> 1)
            y = lax.bitcast_convert_type(yi, jnp.float32)
            for _ in range(3):
                y = y * (1.5 - 0.5 * vv * y * y)
            for j in range(NJ):
                sl = pl.ds(j * L, L)
                rows_v[i, sl] = (xb[j] - mean) * y * g_v[sl] + be_v[sl]
            return icarry

        lax.fori_loop(0, S, token, 0)
        pltpu.sync_copy(rows_v, out_hbm.at[pl.ds(b * S, S)])
        return carry

    lax.fori_loop(0, ROWS_PER_W, chunk, 0)


@jax.jit
def kernel(input_ids, word_emb, pos_emb, type_emb, gamma, beta):
    ids2d = input_ids.astype(jnp.int32).reshape(B * S // 100, 100)
    run = pl.kernel(
        _sc_kernel,
        out_type=jax.ShapeDtypeStruct((B * S, EMBED), jnp.float32),
        mesh=plsc.VectorSubcoreMesh(core_axis_name="c", subcore_axis_name="s"),
        scratch_types=[
            pltpu.VMEM((2, 100), jnp.int32),
            pltpu.VMEM((S, EMBED), jnp.float32),
            pltpu.VMEM((S, EMBED), jnp.float32),
            pltpu.VMEM((1, EMBED), jnp.float32),
            pltpu.VMEM((EMBED,), jnp.float32),
            pltpu.VMEM((EMBED,), jnp.float32),
            pltpu.SemaphoreType.DMA,
        ],
    )
    out = run(ids2d, word_emb, pos_emb, type_emb, gamma, beta)
    return out.reshape(B, S, EMBED)


# speedup vs baseline: 4.7706x; 1.3168x over previous
import functools

import jax
import jax.numpy as jnp
from jax import lax
from jax.experimental import pallas as pl
from jax.experimental.pallas import tpu as pltpu
from jax.experimental.pallas import tpu_sc as plsc

VOCAB = 100000
EMBED = 128
S = 200
B = 1024
EPS = 1e-5

NC, NS, L = 2, 16, 16
NW = NC * NS
ROWS_PER_W = B // NW
NJ = EMBED // L

_GATHER_DNUMS = lax.GatherDimensionNumbers(
    offset_dims=(), collapsed_slice_dims=(0,), start_index_map=(0,))


def _allsum(v):
    lanes = lax.iota(jnp.int32, L)
    for k in (1, 2, 4, 8):
        idx = (lanes ^ k)[:, None]
        v = v + lax.gather(v, idx, dimension_numbers=_GATHER_DNUMS,
                           slice_sizes=(1,),
                           mode=lax.GatherScatterMode.PROMISE_IN_BOUNDS)
    return v


def _sc_kernel(ids_hbm, table_hbm, pos_hbm, type_hbm, gamma_hbm, beta_hbm,
               out_hbm, ids_v, rows_v, bias_v, t_v, g_v, be_v, sem):
    wid = lax.axis_index("s") * NC + lax.axis_index("c")

    pltpu.sync_copy(pos_hbm.at[pl.ds(0, S)], bias_v)
    pltpu.sync_copy(type_hbm.at[pl.ds(0, 1)], t_v)
    pltpu.sync_copy(gamma_hbm, g_v)
    pltpu.sync_copy(beta_hbm, be_v)

    def add_type(r, carry):
        for j in range(NJ):
            sl = pl.ds(j * L, L)
            bias_v[r, sl] = bias_v[r, sl] + t_v[0, sl]
        return carry

    lax.fori_loop(0, S, add_type, 0)

    def chunk(c, carry):
        b = wid * ROWS_PER_W + c
        pltpu.sync_copy(ids_hbm.at[pl.ds(2 * b, 2)], ids_v)
        cp0 = pltpu.async_copy(table_hbm.at[ids_v.at[0]],
                               rows_v.at[pl.ds(0, 100)], sem)
        cp1 = pltpu.async_copy(table_hbm.at[ids_v.at[1]],
                               rows_v.at[pl.ds(100, 100)], sem)
        cp0.wait()
        cp1.wait()

        def one_token(i):
            xb = []
            for j in range(NJ):
                sl = pl.ds(j * L, L)
                xb.append(rows_v[i, sl] + bias_v[i, sl])
            ssum = xb[0]
            for j in range(1, NJ):
                ssum = ssum + xb[j]
            mean = _allsum(ssum) * (1.0 / EMBED)
            ssq = xb[0] * xb[0]
            for j in range(1, NJ):
                ssq = ssq + xb[j] * xb[j]
            var = _allsum(ssq) * (1.0 / EMBED) - mean * mean
            vv = var + EPS
            yi = jnp.int32(0x5F3759DF) - (lax.bitcast_convert_type(vv, jnp.int32) >> 1)
            y = lax.bitcast_convert_type(yi, jnp.float32)
            for _ in range(2):
                y = y * (1.5 - 0.5 * vv * y * y)
            for j in range(NJ):
                sl = pl.ds(j * L, L)
                rows_v[i, sl] = (xb[j] - mean) * y * g_v[sl] + be_v[sl]

        UNROLL = 4
        def token(i, icarry):
            for u in range(UNROLL):
                one_token(i * UNROLL + u)
            return icarry

        lax.fori_loop(0, S // UNROLL, token, 0)
        pltpu.sync_copy(rows_v, out_hbm.at[pl.ds(b * S, S)])
        return carry

    lax.fori_loop(0, ROWS_PER_W, chunk, 0)


@jax.jit
def kernel(input_ids, word_emb, pos_emb, type_emb, gamma, beta):
    ids2d = input_ids.astype(jnp.int32).reshape(B * S // 100, 100)
    run = pl.kernel(
        _sc_kernel,
        out_type=jax.ShapeDtypeStruct((B * S, EMBED), jnp.float32),
        mesh=plsc.VectorSubcoreMesh(core_axis_name="c", subcore_axis_name="s"),
        scratch_types=[
            pltpu.VMEM((2, 100), jnp.int32),
            pltpu.VMEM((S, EMBED), jnp.float32),
            pltpu.VMEM((S, EMBED), jnp.float32),
            pltpu.VMEM((1, EMBED), jnp.float32),
            pltpu.VMEM((EMBED,), jnp.float32),
            pltpu.VMEM((EMBED,), jnp.float32),
            pltpu.SemaphoreType.DMA,
        ],
    )
    out = run(ids2d, word_emb, pos_emb, type_emb, gamma, beta)
    return out.reshape(B, S, EMBED)
